# baseline (speedup 1.0000x reference)
import jax
import jax.numpy as jnp
from jax.experimental import pallas as pl
from jax.experimental.pallas import tpu as pltpu

BATCH = 4096
DIM = 512
NUM_CLASSES = 10000
MARGIN = 0.35

B_TILE = 512
C_TILE = 2048
NB = BATCH // B_TILE
NC = -(-NUM_CLASSES // C_TILE)
LAST_VALID = NUM_CLASSES - (NC - 1) * C_TILE

LOG2E = 1.4426950408889634
LN2 = 0.6931471805599453

LANES = 128
NCHUNK = C_TILE // LANES


def _rownorm(x, scale=1.0):
    inv = scale / (jnp.sqrt(jnp.sum(x * x, axis=1, keepdims=True)) + 1e-12)
    return x * inv


def _chunk_sum(x):
    acc = x[:, :LANES]
    for k in range(1, NCHUNK):
        acc = acc + x[:, k * LANES:(k + 1) * LANES]
    return acc


def _body(f_ref, c_ref, lbl_ref, out_ref, fn_scr, cn_scr, s_scr, t_scr):
    j = pl.program_id(0)
    i = pl.program_id(1)

    @pl.when(j == 0)
    def _():
        fn_scr[i] = _rownorm(f_ref[...], LOG2E)

    @pl.when(i == 0)
    def _():
        cn_scr[...] = _rownorm(c_ref[...])

    logits = jax.lax.dot_general(
        fn_scr[i], cn_scr[...], (((1,), (1,)), ((), ())),
        preferred_element_type=jnp.float32,
    )

    e = jnp.exp2(logits)
    lbl = lbl_ref[0, 0, :]
    cols = j * C_TILE + jax.lax.broadcasted_iota(jnp.int32, (B_TILE, C_TILE), 1)
    masked = jnp.where(cols == lbl[:, None], logits, 0.0)
    t_part = _chunk_sum(masked)

    @pl.when(j == 0)
    def _():
        s_scr[i] = _chunk_sum(e)
        t_scr[i] = t_part

    @pl.when(jnp.logical_and(j > 0, j < NC - 1))
    def _():
        s_scr[i] = s_scr[i] + _chunk_sum(e)
        t_scr[i] = t_scr[i] + t_part

    @pl.when(j == NC - 1)
    def _():
        lane = jax.lax.broadcasted_iota(jnp.int32, (B_TILE, C_TILE), 1)
        ee = jnp.where(lane < LAST_VALID, e, 0.0)
        s128 = s_scr[i] + _chunk_sum(ee)
        t = jnp.sum(t_scr[i] + t_part, axis=1) * LN2
        tm = t - MARGIN
        s = jnp.sum(s128, axis=1) - jnp.exp(t) + jnp.exp(tm)
        out_ref[0, :] = jnp.log(s) - tm


def kernel(feature, label, centers):
    lbl3 = label.reshape(NB, 1, B_TILE)
    out = pl.pallas_call(
        _body,
        grid=(NC, NB),
        in_specs=[
            pl.BlockSpec((B_TILE, DIM), lambda j, i: (i, 0)),
            pl.BlockSpec((C_TILE, DIM), lambda j, i: (j, 0)),
            pl.BlockSpec((1, 1, B_TILE), lambda j, i: (i, 0, 0)),
        ],
        out_specs=pl.BlockSpec((1, B_TILE), lambda j, i: (0, i)),
        out_shape=jax.ShapeDtypeStruct((1, BATCH), jnp.float32),
        scratch_shapes=[
            pltpu.VMEM((NB, B_TILE, DIM), jnp.float32),
            pltpu.VMEM((C_TILE, DIM), jnp.float32),
            pltpu.VMEM((NB, B_TILE, LANES), jnp.float32),
            pltpu.VMEM((NB, B_TILE, LANES), jnp.float32),
        ],
    )(feature, centers, lbl3)
    return out.reshape(BATCH)


# speedup vs baseline: 1.0470x; 1.0470x over previous
import jax
import jax.numpy as jnp
from jax.experimental import pallas as pl
from jax.experimental.pallas import tpu as pltpu

BATCH = 4096
DIM = 512
NUM_CLASSES = 10000
MARGIN = 0.35

B_TILE = 512
C_TILE = 2560
NB = BATCH // B_TILE
NC = -(-NUM_CLASSES // C_TILE)
LAST_VALID = NUM_CLASSES - (NC - 1) * C_TILE

LOG2E = 1.4426950408889634
LN2 = 0.6931471805599453

LANES = 128
NCHUNK = C_TILE // LANES


def _rownorm(x, scale=1.0):
    inv = scale / (jnp.sqrt(jnp.sum(x * x, axis=1, keepdims=True)) + 1e-12)
    return x * inv


def _chunk_sum(x):
    acc = x[:, :LANES]
    for k in range(1, NCHUNK):
        acc = acc + x[:, k * LANES:(k + 1) * LANES]
    return acc


def _body(f_ref, c_ref, lbl_ref, out_ref, fn_scr, cn_scr, s_scr, t_scr):
    j = pl.program_id(0)
    i = pl.program_id(1)

    @pl.when(j == 0)
    def _():
        fn_scr[i] = _rownorm(f_ref[...], LOG2E)

    @pl.when(i == 0)
    def _():
        cn_scr[...] = _rownorm(c_ref[...])

    logits = jax.lax.dot_general(
        fn_scr[i], cn_scr[...], (((1,), (1,)), ((), ())),
        preferred_element_type=jnp.float32,
    )

    e = jnp.exp2(logits)
    lbl = lbl_ref[0, 0, :]
    cols = j * C_TILE + jax.lax.broadcasted_iota(jnp.int32, (B_TILE, C_TILE), 1)
    masked = jnp.where(cols == lbl[:, None], logits, 0.0)
    t_part = _chunk_sum(masked)

    @pl.when(j == 0)
    def _():
        s_scr[i] = _chunk_sum(e)
        t_scr[i] = t_part

    @pl.when(jnp.logical_and(j > 0, j < NC - 1))
    def _():
        s_scr[i] = s_scr[i] + _chunk_sum(e)
        t_scr[i] = t_scr[i] + t_part

    @pl.when(j == NC - 1)
    def _():
        lane = jax.lax.broadcasted_iota(jnp.int32, (B_TILE, C_TILE), 1)
        ee = jnp.where(lane < LAST_VALID, e, 0.0)
        s128 = s_scr[i] + _chunk_sum(ee)
        t = jnp.sum(t_scr[i] + t_part, axis=1) * LN2
        tm = t - MARGIN
        s = jnp.sum(s128, axis=1) - jnp.exp(t) + jnp.exp(tm)
        out_ref[0, :] = jnp.log(s) - tm


def kernel(feature, label, centers):
    lbl3 = label.reshape(NB, 1, B_TILE)
    out = pl.pallas_call(
        _body,
        grid=(NC, NB),
        in_specs=[
            pl.BlockSpec((B_TILE, DIM), lambda j, i: (i, 0)),
            pl.BlockSpec((C_TILE, DIM), lambda j, i: (j, 0)),
            pl.BlockSpec((1, 1, B_TILE), lambda j, i: (i, 0, 0)),
        ],
        out_specs=pl.BlockSpec((1, B_TILE), lambda j, i: (0, i)),
        out_shape=jax.ShapeDtypeStruct((1, BATCH), jnp.float32),
        scratch_shapes=[
            pltpu.VMEM((NB, B_TILE, DIM), jnp.float32),
            pltpu.VMEM((C_TILE, DIM), jnp.float32),
            pltpu.VMEM((NB, B_TILE, LANES), jnp.float32),
            pltpu.VMEM((NB, B_TILE, LANES), jnp.float32),
        ],
    )(feature, centers, lbl3)
    return out.reshape(BATCH)


# speedup vs baseline: 1.1555x; 1.1037x over previous
import jax
import jax.numpy as jnp
from jax.experimental import pallas as pl
from jax.experimental.pallas import tpu as pltpu

BATCH = 4096
DIM = 512
NUM_CLASSES = 10000
MARGIN = 0.35

B_TILE = 512
C_TILE = 5120
NB = BATCH // B_TILE
NC = -(-NUM_CLASSES // C_TILE)
LAST_VALID = NUM_CLASSES - (NC - 1) * C_TILE

LOG2E = 1.4426950408889634
LN2 = 0.6931471805599453

LANES = 128
NCHUNK = C_TILE // LANES


def _rownorm(x, scale=1.0):
    inv = scale / (jnp.sqrt(jnp.sum(x * x, axis=1, keepdims=True)) + 1e-12)
    return x * inv


def _chunk_sum(x):
    acc = x[:, :LANES]
    for k in range(1, NCHUNK):
        acc = acc + x[:, k * LANES:(k + 1) * LANES]
    return acc


def _body(f_ref, c_ref, lbl_ref, out_ref, fn_scr, cn_scr, s_scr, t_scr):
    j = pl.program_id(0)
    i = pl.program_id(1)

    @pl.when(j == 0)
    def _():
        fn_scr[i] = _rownorm(f_ref[...], LOG2E)

    @pl.when(i == 0)
    def _():
        cn_scr[...] = _rownorm(c_ref[...])

    logits = jax.lax.dot_general(
        fn_scr[i], cn_scr[...], (((1,), (1,)), ((), ())),
        preferred_element_type=jnp.float32,
    )

    e = jnp.exp2(logits)
    lbl = lbl_ref[0, 0, :]
    cols = j * C_TILE + jax.lax.broadcasted_iota(jnp.int32, (B_TILE, C_TILE), 1)
    masked = jnp.where(cols == lbl[:, None], logits, 0.0)
    t_part = _chunk_sum(masked)

    @pl.when(j == 0)
    def _():
        s_scr[i] = _chunk_sum(e)
        t_scr[i] = t_part

    @pl.when(jnp.logical_and(j > 0, j < NC - 1))
    def _():
        s_scr[i] = s_scr[i] + _chunk_sum(e)
        t_scr[i] = t_scr[i] + t_part

    @pl.when(j == NC - 1)
    def _():
        lane = jax.lax.broadcasted_iota(jnp.int32, (B_TILE, C_TILE), 1)
        ee = jnp.where(lane < LAST_VALID, e, 0.0)
        s128 = s_scr[i] + _chunk_sum(ee)
        t = jnp.sum(t_scr[i] + t_part, axis=1) * LN2
        tm = t - MARGIN
        s = jnp.sum(s128, axis=1) - jnp.exp(t) + jnp.exp(tm)
        out_ref[0, :] = jnp.log(s) - tm


def kernel(feature, label, centers):
    lbl3 = label.reshape(NB, 1, B_TILE)
    out = pl.pallas_call(
        _body,
        grid=(NC, NB),
        in_specs=[
            pl.BlockSpec((B_TILE, DIM), lambda j, i: (i, 0)),
            pl.BlockSpec((C_TILE, DIM), lambda j, i: (j, 0)),
            pl.BlockSpec((1, 1, B_TILE), lambda j, i: (i, 0, 0)),
        ],
        out_specs=pl.BlockSpec((1, B_TILE), lambda j, i: (0, i)),
        out_shape=jax.ShapeDtypeStruct((1, BATCH), jnp.float32),
        scratch_shapes=[
            pltpu.VMEM((NB, B_TILE, DIM), jnp.float32),
            pltpu.VMEM((C_TILE, DIM), jnp.float32),
            pltpu.VMEM((NB, B_TILE, LANES), jnp.float32),
            pltpu.VMEM((NB, B_TILE, LANES), jnp.float32),
        ],
    )(feature, centers, lbl3)
    return out.reshape(BATCH)
